# baseline (device time: 21907 ns/iter reference)
import jax
import jax.numpy as jnp
from jax import lax
from jax.experimental import pallas as pl
from jax.experimental.pallas import tpu as pltpu

N_DEV = 4


def kernel(A, B):
    m, k = A.shape
    _, n = B.shape
    ch = m // N_DEV

    def body(a_ref, b_ref, out_ref, partial_ref, rs_buf,
             rs_send, rs_recv, ag_send, ag_recv):
        me = lax.axis_index("i")

        bar = pltpu.get_barrier_semaphore()
        for d in range(1, N_DEV):
            pl.semaphore_signal(
                bar, inc=1,
                device_id=((me + d) % N_DEV,),
                device_id_type=pl.DeviceIdType.MESH,
            )
        pl.semaphore_wait(bar, N_DEV - 1)

        rs = {}
        for d in (2, 1, 3):
            tgt = (me + d) % N_DEV
            partial_ref[pl.ds(tgt * ch, ch), :] = jnp.dot(
                a_ref[pl.ds(tgt * ch, ch), :], b_ref[:, :],
                preferred_element_type=jnp.float32,
            )
            rdma = pltpu.make_async_remote_copy(
                src_ref=partial_ref.at[pl.ds(tgt * ch, ch), :],
                dst_ref=rs_buf.at[d - 1],
                send_sem=rs_send.at[d - 1],
                recv_sem=rs_recv.at[d - 1],
                device_id=(tgt,),
                device_id_type=pl.DeviceIdType.MESH,
            )
            rdma.start()
            rs[d] = rdma
        acc = jnp.dot(
            a_ref[pl.ds(me * ch, ch), :], b_ref[:, :],
            preferred_element_type=jnp.float32,
        )
        for d in (1, 3, 2):
            rs[d].wait_recv()
            acc = acc + rs_buf[d - 1]

        out_ref[pl.ds(me * ch, ch), :] = acc
        ag = []
        for d in (2, 1, 3):
            tgt = (me + d) % N_DEV
            rdma = pltpu.make_async_remote_copy(
                src_ref=out_ref.at[pl.ds(me * ch, ch), :],
                dst_ref=out_ref.at[pl.ds(me * ch, ch), :],
                send_sem=ag_send.at[d - 1],
                recv_sem=ag_recv.at[d - 1],
                device_id=(tgt,),
                device_id_type=pl.DeviceIdType.MESH,
            )
            rdma.start()
            ag.append(rdma)
        for d in (1, 3, 2):
            rs[d].wait_send()
        for r in ag:
            r.wait()

    return pl.pallas_call(
        body,
        out_shape=jax.ShapeDtypeStruct((m, n), jnp.float32),
        in_specs=[
            pl.BlockSpec(memory_space=pltpu.VMEM),
            pl.BlockSpec(memory_space=pltpu.VMEM),
        ],
        out_specs=pl.BlockSpec(memory_space=pltpu.VMEM),
        scratch_shapes=[
            pltpu.VMEM((m, n), jnp.float32),
            pltpu.VMEM((N_DEV - 1, ch, n), jnp.float32),
            pltpu.SemaphoreType.DMA((N_DEV - 1,)),
            pltpu.SemaphoreType.DMA((N_DEV - 1,)),
            pltpu.SemaphoreType.DMA((N_DEV - 1,)),
            pltpu.SemaphoreType.DMA((N_DEV - 1,)),
        ],
        compiler_params=pltpu.CompilerParams(collective_id=0),
    )(A, B)


# device time: 19770 ns/iter; 1.1081x vs baseline; 1.1081x over previous
import jax
import jax.numpy as jnp
from jax import lax
from jax.experimental import pallas as pl
from jax.experimental.pallas import tpu as pltpu

N_DEV = 4
N_HALF = 2


def kernel(A, B):
    m, k = A.shape
    _, n = B.shape
    ch = m // N_DEV
    nh = n // N_HALF

    def body(a_ref, b_ref, out_ref, partial_ref, rs_buf,
             rs_send, rs_recv, ag_send, ag_recv):
        me = lax.axis_index("i")

        bar = pltpu.get_barrier_semaphore()
        for d in range(1, N_DEV):
            pl.semaphore_signal(
                bar, inc=1,
                device_id=((me + d) % N_DEV,),
                device_id_type=pl.DeviceIdType.MESH,
            )
        pl.semaphore_wait(bar, N_DEV - 1)

        rs = {}
        for h in range(N_HALF):
            partial_ref[:, pl.ds(h * nh, nh)] = jnp.dot(
                a_ref[:, :], b_ref[:, pl.ds(h * nh, nh)],
                preferred_element_type=jnp.float32,
            )
            for d in (2, 1, 3):
                tgt = (me + d) % N_DEV
                rdma = pltpu.make_async_remote_copy(
                    src_ref=partial_ref.at[pl.ds(tgt * ch, ch),
                                           pl.ds(h * nh, nh)],
                    dst_ref=rs_buf.at[h, d - 1],
                    send_sem=rs_send.at[h, d - 1],
                    recv_sem=rs_recv.at[h, d - 1],
                    device_id=(tgt,),
                    device_id_type=pl.DeviceIdType.MESH,
                )
                rdma.start()
                rs[(h, d)] = rdma

        ag = []
        for h in range(N_HALF):
            for d in (1, 3, 2):
                rs[(h, d)].wait_recv()
            out_ref[pl.ds(me * ch, ch), pl.ds(h * nh, nh)] = (
                partial_ref[pl.ds(me * ch, ch), pl.ds(h * nh, nh)]
                + rs_buf[h, 0] + rs_buf[h, 1] + rs_buf[h, 2]
            )
            for d in (2, 1, 3):
                tgt = (me + d) % N_DEV
                rdma = pltpu.make_async_remote_copy(
                    src_ref=out_ref.at[pl.ds(me * ch, ch),
                                       pl.ds(h * nh, nh)],
                    dst_ref=out_ref.at[pl.ds(me * ch, ch),
                                       pl.ds(h * nh, nh)],
                    send_sem=ag_send.at[h, d - 1],
                    recv_sem=ag_recv.at[h, d - 1],
                    device_id=(tgt,),
                    device_id_type=pl.DeviceIdType.MESH,
                )
                rdma.start()
                ag.append(rdma)

        for r in rs.values():
            r.wait_send()
        for r in ag:
            r.wait()

    return pl.pallas_call(
        body,
        out_shape=jax.ShapeDtypeStruct((m, n), jnp.float32),
        in_specs=[
            pl.BlockSpec(memory_space=pltpu.VMEM),
            pl.BlockSpec(memory_space=pltpu.VMEM),
        ],
        out_specs=pl.BlockSpec(memory_space=pltpu.VMEM),
        scratch_shapes=[
            pltpu.VMEM((m, n), jnp.float32),
            pltpu.VMEM((N_HALF, N_DEV - 1, ch, nh), jnp.float32),
            pltpu.SemaphoreType.DMA((N_HALF, N_DEV - 1)),
            pltpu.SemaphoreType.DMA((N_HALF, N_DEV - 1)),
            pltpu.SemaphoreType.DMA((N_HALF, N_DEV - 1)),
            pltpu.SemaphoreType.DMA((N_HALF, N_DEV - 1)),
        ],
        compiler_params=pltpu.CompilerParams(collective_id=0),
    )(A, B)
